# baseline (speedup 1.0000x reference)
import functools
import math

import jax
import jax.numpy as jnp
from jax import lax
from jax.experimental import pallas as pl
from jax.experimental.pallas import tpu as pltpu
from jax.experimental.pallas import tpu_sc as plsc

_B, _N, _M, _K = 4, 8192, 2048, 64
_INFEA = 16
_DIST2 = 0.4 * 0.4
_EPS = 1e-5
_C = 32
_MB = 32
_PB = _MB * _K
_P_TOT = _B * _M * _K


def _x_tile(g, qpad):
    qb = jnp.broadcast_to(qpad[:, None, :], (_MB, _K, _C)).reshape(_PB, _C)
    return g - qb


def _stats1_kernel(g_ref, npc_ref, acc_ref):
    b = pl.program_id(0)
    mi = pl.program_id(1)

    @pl.when(jnp.logical_and(b == 0, mi == 0))
    def _():
        acc_ref[...] = jnp.zeros_like(acc_ref)

    x = _x_tile(g_ref[...], npc_ref[0])
    gram = jax.lax.dot_general(x, x, (((0,), (0,)), ((), ())),
                               preferred_element_type=jnp.float32)
    s1 = jnp.sum(x, axis=0)
    acc_ref[0:_C, :] += gram
    acc_ref[_C:_C + 1, :] += s1[None, :]


def _stats2_kernel(g_ref, npc_ref, a1_ref, c1_ref, acc_ref):
    b = pl.program_id(0)
    mi = pl.program_id(1)

    @pl.when(jnp.logical_and(b == 0, mi == 0))
    def _():
        acc_ref[...] = jnp.zeros_like(acc_ref)

    x = _x_tile(g_ref[...], npc_ref[0])
    h1 = jnp.maximum(
        jax.lax.dot_general(x, a1_ref[...], (((1,), (1,)), ((), ())),
                            preferred_element_type=jnp.float32)
        + c1_ref[...], 0.0)
    gram = jax.lax.dot_general(h1, h1, (((0,), (0,)), ((), ())),
                               preferred_element_type=jnp.float32)
    acc_ref[0:_C, :] += gram
    acc_ref[_C:_C + 1, :] += jnp.sum(h1, axis=0)[None, :]


def _stats3_kernel(g_ref, npc_ref, a1_ref, c1_ref, a2_ref, c2_ref, acc_ref):
    b = pl.program_id(0)
    mi = pl.program_id(1)

    @pl.when(jnp.logical_and(b == 0, mi == 0))
    def _():
        acc_ref[...] = jnp.zeros_like(acc_ref)

    x = _x_tile(g_ref[...], npc_ref[0])
    h1 = jnp.maximum(
        jax.lax.dot_general(x, a1_ref[...], (((1,), (1,)), ((), ())),
                            preferred_element_type=jnp.float32)
        + c1_ref[...], 0.0)
    h2 = jnp.maximum(
        jax.lax.dot_general(h1, a2_ref[...], (((1,), (1,)), ((), ())),
                            preferred_element_type=jnp.float32)
        + c2_ref[...], 0.0)
    gram = jax.lax.dot_general(h2, h2, (((0,), (0,)), ((), ())),
                               preferred_element_type=jnp.float32)
    acc_ref[0:_C, :] += gram
    acc_ref[_C:_C + 1, :] += jnp.sum(h2, axis=0)[None, :]


def _final_kernel(g_ref, npc_ref, a1_ref, c1_ref, a2_ref, c2_ref,
                  a3_ref, c3_ref, valid_ref, out_ref):
    b = pl.program_id(0)
    x = _x_tile(g_ref[...], npc_ref[0])
    h1 = jnp.maximum(
        jax.lax.dot_general(x, a1_ref[...], (((1,), (1,)), ((), ())),
                            preferred_element_type=jnp.float32)
        + c1_ref[...], 0.0)
    h2 = jnp.maximum(
        jax.lax.dot_general(h1, a2_ref[...], (((1,), (1,)), ((), ())),
                            preferred_element_type=jnp.float32)
        + c2_ref[...], 0.0)
    y = jnp.maximum(
        jax.lax.dot_general(h2, a3_ref[...], (((1,), (1,)), ((), ())),
                            preferred_element_type=jnp.float32)
        + c3_ref[...], 0.0)
    vrow = valid_ref[0, 0]
    vmask = jnp.broadcast_to(vrow.reshape(_MB, 1, 1), (_MB, _K, 1))
    y = y * vmask.reshape(_PB, 1)
    out_ref[0] = y.T.reshape(64, _MB, _K)


def _fold(acc, W, bvec, gvec, beta, cin):
    n = float(_P_TOT)
    gram = acc[0:_C, 0:_C] / n
    mu = acc[_C, 0:_C] / n
    Wp = jnp.zeros((W.shape[0], _C), jnp.float32).at[:, :cin].set(W)
    wmu = Wp @ mu
    mean_y = wmu + bvec
    e_yy = jnp.einsum('oc,cd,od->o', Wp, gram, Wp) + 2.0 * bvec * wmu + bvec * bvec
    var_y = e_yy - mean_y * mean_y
    a = gvec * jax.lax.rsqrt(var_y + _EPS)
    A = a[:, None] * Wp
    c = a * bvec + beta - a * mean_y
    return A, c[None, :]


def _mlp_passes(G, npc32, valid, W1, b1, g1, beta1, W2, b2, g2, beta2,
                W3, b3, g3, beta3):
    valid = valid.reshape(_B, _M // _MB, 1, _MB)
    grid = (_B, _M // _MB)
    g_spec = pl.BlockSpec((_PB, _C), lambda b, mi: (b * (_M // _MB) + mi, 0))
    npc_spec = pl.BlockSpec((1, _MB, _C), lambda b, mi: (b, mi, 0))
    acc_shape = jax.ShapeDtypeStruct((_C + 8, _C), jnp.float32)
    acc_spec = pl.BlockSpec((_C + 8, _C), lambda b, mi: (0, 0))
    mat_spec = pl.BlockSpec((_C, _C), lambda b, mi: (0, 0))
    c_spec = pl.BlockSpec((1, _C), lambda b, mi: (0, 0))

    acc1 = pl.pallas_call(
        _stats1_kernel, grid=grid,
        in_specs=[g_spec, npc_spec],
        out_specs=acc_spec, out_shape=acc_shape,
    )(G, npc32)
    A1, c1 = _fold(acc1, W1, b1, g1, beta1, 3 + _INFEA)

    acc2 = pl.pallas_call(
        _stats2_kernel, grid=grid,
        in_specs=[g_spec, npc_spec, mat_spec, c_spec],
        out_specs=acc_spec, out_shape=acc_shape,
    )(G, npc32, A1, c1)
    A2, c2 = _fold(acc2, W2, b2, g2, beta2, 32)

    acc3 = pl.pallas_call(
        _stats3_kernel, grid=grid,
        in_specs=[g_spec, npc_spec, mat_spec, c_spec, mat_spec, c_spec],
        out_specs=acc_shape and acc_spec, out_shape=acc_shape,
    )(G, npc32, A1, c1, A2, c2)
    A3, c3 = _fold(acc3, W3, b3, g3, beta3, 32)
    A3p = jnp.zeros((64, _C), jnp.float32).at[:, :].set(A3)

    out = pl.pallas_call(
        _final_kernel, grid=grid,
        in_specs=[g_spec, npc_spec, mat_spec, c_spec, mat_spec, c_spec,
                  pl.BlockSpec((64, _C), lambda b, mi: (0, 0)),
                  pl.BlockSpec((1, 64), lambda b, mi: (0, 0)),
                  pl.BlockSpec((1, 1, 1, _MB), lambda b, mi: (b, mi, 0, 0))],
        out_specs=pl.BlockSpec((1, 64, _MB, _K), lambda b, mi: (b, 0, mi, 0)),
        out_shape=jax.ShapeDtypeStruct((_B, 64, _M, _K), jnp.float32),
    )(G, npc32, A1, c1, A2, c2, A3p, c3, valid)
    return out


_NB = 2048


def _prep_kernel(pc_ref, feat_ref, p_ref):
    cat = jnp.concatenate(
        [pc_ref[0], feat_ref[0],
         jnp.zeros((_C - 3 - _INFEA, _NB), jnp.float32)], axis=0)
    p_ref[...] = cat.T


def _build_point_table(pc, feat):
    grid = (_B, _N // _NB)
    return pl.pallas_call(
        _prep_kernel, grid=grid,
        in_specs=[pl.BlockSpec((1, 3, _NB), lambda b, ni: (b, 0, ni)),
                  pl.BlockSpec((1, _INFEA, _NB), lambda b, ni: (b, 0, ni))],
        out_specs=pl.BlockSpec((_NB, _C), lambda b, ni: (b * (_N // _NB) + ni, 0)),
        out_shape=jax.ShapeDtypeStruct((_B * _N, _C), jnp.float32),
    )(pc, feat)


def _pack_kernel(c_ref, out_ref):
    x, y, z = c_ref[0, 0], c_ref[0, 1], c_ref[0, 2]
    r = c_ref[0].astype(jnp.bfloat16).astype(jnp.float32)
    s = (x * x + y * y) + z * z
    out_ref[0] = jnp.concatenate([r, s[None, :]], axis=0)


def _pack4(arr, L):
    nb = min(L, 2048)
    grid = (_B, L // nb)
    out = pl.pallas_call(
        _pack_kernel, grid=grid,
        in_specs=[pl.BlockSpec((1, 3, nb), lambda b, ni: (b, 0, ni))],
        out_specs=pl.BlockSpec((1, 4, nb), lambda b, ni: (b, 0, ni)),
        out_shape=jax.ShapeDtypeStruct((_B, 4, L), jnp.float32),
    )(arr)
    return out.reshape(_B * 4, L)


_NC, _NS = 2, 16
_NW = _NC * _NS
_CPW = (_B * _M) // _NW
_SEG = _M // (_NW // _B)
_NCHUNK = _N // 16


def _sc_query_gather(pc, new_pc, ptab):
    mesh = plsc.VectorSubcoreMesh(core_axis_name="c", subcore_axis_name="s")

    @functools.partial(
        pl.kernel,
        out_type=(jax.ShapeDtypeStruct((_P_TOT, _C), jnp.float32),
                  jax.ShapeDtypeStruct((_B * _M,), jnp.float32)),
        mesh=mesh,
        compiler_params=pltpu.CompilerParams(needs_layout_passes=False,
                                             use_tc_tiling_on_sc=False),
        scratch_types=[
            pltpu.VMEM((4 * _N,), jnp.float32),
            pltpu.VMEM((4 * _SEG,), jnp.float32),
            pltpu.VMEM((96,), jnp.int32),
            pltpu.VMEM((_K,), jnp.int32),
            pltpu.VMEM((_K, _C), jnp.float32),
            pltpu.VMEM((_SEG,), jnp.float32),
            pltpu.SemaphoreType.DMA,
        ],
    )
    def sck(pc_hbm, npc_hbm, ptab_hbm, g_hbm, valid_hbm,
            pcx, npcs, idxbuf, gidx, rows, flags, sem):
        wid = lax.axis_index("s") * _NC + lax.axis_index("c")
        b = wid // (_NW // _B)
        seg = wid % (_NW // _B)
        m0 = seg * _SEG
        bn = b * _N
        for r in range(4):
            pltpu.sync_copy(pc_hbm.at[b * 4 + r], pcx.at[pl.ds(r * _N, _N)])
            pltpu.sync_copy(npc_hbm.at[b * 4 + r, pl.ds(m0, _SEG)],
                            npcs.at[pl.ds(r * _SEG, _SEG)])
        iota = lax.iota(jnp.int32, 16)
        lane0 = iota == 0
        zeros16 = jnp.zeros((16,), jnp.int32)

        def per_centroid(mi, _):
            mi16 = jnp.full((16,), mi, jnp.int32)
            qx = plsc.load_gather(npcs, [mi16])
            qy = plsc.load_gather(npcs, [mi16 + _SEG])
            qz = plsc.load_gather(npcs, [mi16 + 2 * _SEG])
            sq = plsc.load_gather(npcs, [mi16 + 3 * _SEG])
            for j in range(6):
                idxbuf[pl.ds(j * 16, 16)] = zeros16

            def scan_cond(state):
                nc, off = state
                return jnp.logical_and(nc < _NCHUNK, off < _K)

            def scan_body(state):
                nc, off = state
                n0 = nc * 16
                px = pcx[pl.ds(n0, 16)]
                py = pcx[pl.ds(_N + n0, 16)]
                pz = pcx[pl.ds(2 * _N + n0, 16)]
                sp = pcx[pl.ds(3 * _N + n0, 16)]
                dot = px * qx + py * qy + pz * qz
                d2 = sq + sp - 2.0 * dot
                msk = d2 < _DIST2

                def found(off):
                    m32 = msk.astype(jnp.int32)
                    r = plsc.cumsum(m32)
                    smask = jnp.logical_and(msk, (r + off) <= _K)
                    plsc.store_compressed(idxbuf.at[pl.ds(off, 16)],
                                          iota + n0, mask=smask)
                    return jnp.minimum(off + jnp.sum(m32), _K)

                off = lax.cond(jnp.any(msk), found, lambda o: o, off)
                return nc + 1, off

            _, total = lax.while_loop(scan_cond, scan_body,
                                      (jnp.int32(0), jnp.int32(0)))

            for j in range(_K // 16):
                gidx[pl.ds(j * 16, 16)] = idxbuf[pl.ds(j * 16, 16)] + bn
            pltpu.async_copy(ptab_hbm.at[gidx], rows, sem).wait()
            rowbase = (b * _M + m0 + mi) * _K
            pltpu.sync_copy(rows, g_hbm.at[pl.ds(rowbase, _K)])
            flagv = jnp.where(jnp.full((16,), total) > 0, 1.0, 0.0)
            plsc.store_scatter(flags, [jnp.full((16,), mi, jnp.int32)],
                               flagv, mask=lane0)
            return 0

        lax.fori_loop(0, _SEG, per_centroid, 0)
        pltpu.sync_copy(flags, valid_hbm.at[pl.ds(b * _M + m0, _SEG)])

    return sck(_pack4(pc, _N), _pack4(new_pc, _M), ptab)


def kernel(pc, feat, new_pc, W1, b1, g1, beta1, W2, b2, g2, beta2,
           W3, b3, g3, beta3):
    ptab = _build_point_table(pc, feat)
    G, validf = _sc_query_gather(pc, new_pc, ptab)
    valid = validf.reshape(_B, _M)

    npc32 = jnp.zeros((_B, _M, _C), jnp.float32).at[:, :, :3].set(
        jnp.moveaxis(new_pc, 1, 2))

    return _mlp_passes(G, npc32, valid, W1, b1, g1, beta1,
                       W2, b2, g2, beta2, W3, b3, g3, beta3)


# speedup vs baseline: 2.4559x; 2.4559x over previous
import functools
import math

import jax
import jax.numpy as jnp
from jax import lax
from jax.experimental import pallas as pl
from jax.experimental.pallas import tpu as pltpu
from jax.experimental.pallas import tpu_sc as plsc

_B, _N, _M, _K = 4, 8192, 2048, 64
_INFEA = 16
_DIST2 = 0.4 * 0.4
_EPS = 1e-5
_C = 32
_MB = 32
_PB = _MB * _K
_P_TOT = _B * _M * _K


def _x_tile(g, qpad):
    qb = jnp.broadcast_to(qpad[:, None, :], (_MB, _K, _C)).reshape(_PB, _C)
    return g - qb


def _stats1_kernel(g_ref, npc_ref, acc_ref):
    b = pl.program_id(0)
    mi = pl.program_id(1)

    @pl.when(jnp.logical_and(b == 0, mi == 0))
    def _():
        acc_ref[...] = jnp.zeros_like(acc_ref)

    x = _x_tile(g_ref[...], npc_ref[0])
    gram = jax.lax.dot_general(x, x, (((0,), (0,)), ((), ())),
                               preferred_element_type=jnp.float32)
    s1 = jnp.sum(x, axis=0)
    acc_ref[0:_C, :] += gram
    acc_ref[_C:_C + 1, :] += s1[None, :]


def _stats2_kernel(g_ref, npc_ref, a1_ref, c1_ref, acc_ref):
    b = pl.program_id(0)
    mi = pl.program_id(1)

    @pl.when(jnp.logical_and(b == 0, mi == 0))
    def _():
        acc_ref[...] = jnp.zeros_like(acc_ref)

    x = _x_tile(g_ref[...], npc_ref[0])
    h1 = jnp.maximum(
        jax.lax.dot_general(x, a1_ref[...], (((1,), (1,)), ((), ())),
                            preferred_element_type=jnp.float32)
        + c1_ref[...], 0.0)
    gram = jax.lax.dot_general(h1, h1, (((0,), (0,)), ((), ())),
                               preferred_element_type=jnp.float32)
    acc_ref[0:_C, :] += gram
    acc_ref[_C:_C + 1, :] += jnp.sum(h1, axis=0)[None, :]


def _stats3_kernel(g_ref, npc_ref, a1_ref, c1_ref, a2_ref, c2_ref, acc_ref):
    b = pl.program_id(0)
    mi = pl.program_id(1)

    @pl.when(jnp.logical_and(b == 0, mi == 0))
    def _():
        acc_ref[...] = jnp.zeros_like(acc_ref)

    x = _x_tile(g_ref[...], npc_ref[0])
    h1 = jnp.maximum(
        jax.lax.dot_general(x, a1_ref[...], (((1,), (1,)), ((), ())),
                            preferred_element_type=jnp.float32)
        + c1_ref[...], 0.0)
    h2 = jnp.maximum(
        jax.lax.dot_general(h1, a2_ref[...], (((1,), (1,)), ((), ())),
                            preferred_element_type=jnp.float32)
        + c2_ref[...], 0.0)
    gram = jax.lax.dot_general(h2, h2, (((0,), (0,)), ((), ())),
                               preferred_element_type=jnp.float32)
    acc_ref[0:_C, :] += gram
    acc_ref[_C:_C + 1, :] += jnp.sum(h2, axis=0)[None, :]


def _final_kernel(g_ref, npc_ref, a1_ref, c1_ref, a2_ref, c2_ref,
                  a3_ref, c3_ref, valid_ref, out_ref):
    b = pl.program_id(0)
    x = _x_tile(g_ref[...], npc_ref[0])
    h1 = jnp.maximum(
        jax.lax.dot_general(x, a1_ref[...], (((1,), (1,)), ((), ())),
                            preferred_element_type=jnp.float32)
        + c1_ref[...], 0.0)
    h2 = jnp.maximum(
        jax.lax.dot_general(h1, a2_ref[...], (((1,), (1,)), ((), ())),
                            preferred_element_type=jnp.float32)
        + c2_ref[...], 0.0)
    y = jnp.maximum(
        jax.lax.dot_general(h2, a3_ref[...], (((1,), (1,)), ((), ())),
                            preferred_element_type=jnp.float32)
        + c3_ref[...], 0.0)
    vrow = valid_ref[0, 0]
    vmask = jnp.broadcast_to(vrow.reshape(_MB, 1, 1), (_MB, _K, 1))
    y = y * vmask.reshape(_PB, 1)
    out_ref[0] = y.T.reshape(64, _MB, _K)


def _fold(acc, W, bvec, gvec, beta, cin):
    n = float(_P_TOT)
    gram = acc[0:_C, 0:_C] / n
    mu = acc[_C, 0:_C] / n
    Wp = jnp.zeros((W.shape[0], _C), jnp.float32).at[:, :cin].set(W)
    wmu = Wp @ mu
    mean_y = wmu + bvec
    e_yy = jnp.einsum('oc,cd,od->o', Wp, gram, Wp) + 2.0 * bvec * wmu + bvec * bvec
    var_y = e_yy - mean_y * mean_y
    a = gvec * jax.lax.rsqrt(var_y + _EPS)
    A = a[:, None] * Wp
    c = a * bvec + beta - a * mean_y
    return A, c[None, :]


def _mlp_passes(G, npc32, valid, W1, b1, g1, beta1, W2, b2, g2, beta2,
                W3, b3, g3, beta3):
    valid = valid.reshape(_B, _M // _MB, 1, _MB)
    grid = (_B, _M // _MB)
    g_spec = pl.BlockSpec((_PB, _C), lambda b, mi: (b * (_M // _MB) + mi, 0))
    npc_spec = pl.BlockSpec((1, _MB, _C), lambda b, mi: (b, mi, 0))
    acc_shape = jax.ShapeDtypeStruct((_C + 8, _C), jnp.float32)
    acc_spec = pl.BlockSpec((_C + 8, _C), lambda b, mi: (0, 0))
    mat_spec = pl.BlockSpec((_C, _C), lambda b, mi: (0, 0))
    c_spec = pl.BlockSpec((1, _C), lambda b, mi: (0, 0))

    acc1 = pl.pallas_call(
        _stats1_kernel, grid=grid,
        in_specs=[g_spec, npc_spec],
        out_specs=acc_spec, out_shape=acc_shape,
    )(G, npc32)
    A1, c1 = _fold(acc1, W1, b1, g1, beta1, 3 + _INFEA)

    acc2 = pl.pallas_call(
        _stats2_kernel, grid=grid,
        in_specs=[g_spec, npc_spec, mat_spec, c_spec],
        out_specs=acc_spec, out_shape=acc_shape,
    )(G, npc32, A1, c1)
    A2, c2 = _fold(acc2, W2, b2, g2, beta2, 32)

    acc3 = pl.pallas_call(
        _stats3_kernel, grid=grid,
        in_specs=[g_spec, npc_spec, mat_spec, c_spec, mat_spec, c_spec],
        out_specs=acc_shape and acc_spec, out_shape=acc_shape,
    )(G, npc32, A1, c1, A2, c2)
    A3, c3 = _fold(acc3, W3, b3, g3, beta3, 32)
    A3p = jnp.zeros((64, _C), jnp.float32).at[:, :].set(A3)

    out = pl.pallas_call(
        _final_kernel, grid=grid,
        in_specs=[g_spec, npc_spec, mat_spec, c_spec, mat_spec, c_spec,
                  pl.BlockSpec((64, _C), lambda b, mi: (0, 0)),
                  pl.BlockSpec((1, 64), lambda b, mi: (0, 0)),
                  pl.BlockSpec((1, 1, 1, _MB), lambda b, mi: (b, mi, 0, 0))],
        out_specs=pl.BlockSpec((1, 64, _MB, _K), lambda b, mi: (b, 0, mi, 0)),
        out_shape=jax.ShapeDtypeStruct((_B, 64, _M, _K), jnp.float32),
    )(G, npc32, A1, c1, A2, c2, A3p, c3, valid)
    return out


_NB = 2048


def _prep_kernel(pc_ref, feat_ref, p_ref):
    cat = jnp.concatenate(
        [pc_ref[0], feat_ref[0],
         jnp.zeros((_C - 3 - _INFEA, _NB), jnp.float32)], axis=0)
    p_ref[...] = cat.T


def _build_point_table(pc, feat):
    grid = (_B, _N // _NB)
    return pl.pallas_call(
        _prep_kernel, grid=grid,
        in_specs=[pl.BlockSpec((1, 3, _NB), lambda b, ni: (b, 0, ni)),
                  pl.BlockSpec((1, _INFEA, _NB), lambda b, ni: (b, 0, ni))],
        out_specs=pl.BlockSpec((_NB, _C), lambda b, ni: (b * (_N // _NB) + ni, 0)),
        out_shape=jax.ShapeDtypeStruct((_B * _N, _C), jnp.float32),
    )(pc, feat)


def _pack_kernel(c_ref, out_ref):
    x, y, z = c_ref[0, 0], c_ref[0, 1], c_ref[0, 2]
    r = c_ref[0].astype(jnp.bfloat16).astype(jnp.float32)
    s = (x * x + y * y) + z * z
    out_ref[0] = jnp.concatenate([r, s[None, :]], axis=0)


def _pack4(arr, L):
    nb = min(L, 2048)
    grid = (_B, L // nb)
    return pl.pallas_call(
        _pack_kernel, grid=grid,
        in_specs=[pl.BlockSpec((1, 3, nb), lambda b, ni: (b, 0, ni))],
        out_specs=pl.BlockSpec((1, 4, nb), lambda b, ni: (b, 0, ni)),
        out_shape=jax.ShapeDtypeStruct((_B, 4, L), jnp.float32),
    )(arr)


_TCH = _N // 16
_LW = 80


_LMB = 128


def _list_kernel(npc4_ref, pc4_ref, e2_ref, out_ref):
    q = npc4_ref[0]
    p = pc4_ref[0]
    dot = jax.lax.dot_general(
        q[:3].T.astype(jnp.bfloat16), p[:3].astype(jnp.bfloat16),
        (((1,), (0,)), ((), ())), preferred_element_type=jnp.float32)
    d2 = q[3][:, None] + p[3][None, :] - 2.0 * dot
    mask01 = (d2 < _DIST2).astype(jnp.bfloat16)
    cnts = jax.lax.dot_general(mask01, e2_ref[...],
                               (((1,), (0,)), ((), ())),
                               preferred_element_type=jnp.float32)
    ti = jax.lax.broadcasted_iota(jnp.int32, (_TCH, _TCH), 0)
    tj = jax.lax.broadcasted_iota(jnp.int32, (_TCH, _TCH), 1)
    tri_excl = (ti < tj).astype(jnp.float32)
    tri_incl = (ti <= tj).astype(jnp.float32)
    cum_excl = jax.lax.dot_general(cnts, tri_excl, (((1,), (0,)), ((), ())),
                                   preferred_element_type=jnp.float32)
    nz = (cnts > 0.0).astype(jnp.float32)
    proc = nz * (cum_excl < float(_K)).astype(jnp.float32)
    cum_proc = jax.lax.dot_general(proc, tri_incl, (((1,), (0,)), ((), ())),
                                   preferred_element_type=jnp.float32)
    jslab = 8
    jj0 = jax.lax.broadcasted_iota(jnp.int32, (_LMB, jslab, _TCH), 1).astype(jnp.float32)
    pieces = []
    for jc in range(_K // jslab):
        jj = jj0 + float(jc * jslab)
        pieces.append(jnp.sum((cum_proc[:, None, :] <= jj).astype(jnp.float32),
                              axis=2))
    ids = jnp.concatenate(pieces, axis=1)
    ids = jnp.minimum(ids, float(_TCH - 1)).astype(jnp.int32)
    nproc = jnp.sum(proc, axis=1).astype(jnp.int32)
    pad = jnp.zeros((_LMB, _LW - _K - 1), jnp.int32)
    out_ref[...] = jnp.concatenate([ids, nproc[:, None], pad], axis=1)


def _build_lists(npc4, pc4):
    e2 = (jnp.arange(_N, dtype=jnp.int32)[:, None] // 16
          == jnp.arange(_TCH, dtype=jnp.int32)[None, :]).astype(jnp.bfloat16)
    grid = (_B, _M // _LMB)
    return pl.pallas_call(
        _list_kernel, grid=grid,
        in_specs=[pl.BlockSpec((1, 4, _LMB), lambda b, mi: (b, 0, mi)),
                  pl.BlockSpec((1, 4, _N), lambda b, mi: (b, 0, 0)),
                  pl.BlockSpec((_N, _TCH), lambda b, mi: (0, 0))],
        out_specs=pl.BlockSpec((_LMB, _LW),
                               lambda b, mi: (b * (_M // _LMB) + mi, 0)),
        out_shape=jax.ShapeDtypeStruct((_B * _M, _LW), jnp.int32),
    )(npc4, pc4, e2)


_NC, _NS = 2, 16
_NW = _NC * _NS
_CPW = (_B * _M) // _NW
_SEG = _M // (_NW // _B)
_NCHUNK = _N // 16


def _sc_query_gather(pc, new_pc, ptab):
    npc4 = _pack4(new_pc, _M)
    pc4 = _pack4(pc, _N)
    p4v = pc4.reshape(_B * 4 * _TCH, 16)
    lists = _build_lists(npc4, pc4)
    mesh = plsc.VectorSubcoreMesh(core_axis_name="c", subcore_axis_name="s")

    @functools.partial(
        pl.kernel,
        out_type=(jax.ShapeDtypeStruct((_P_TOT, _C), jnp.float32),
                  jax.ShapeDtypeStruct((_B * _M,), jnp.float32)),
        mesh=mesh,
        compiler_params=pltpu.CompilerParams(needs_layout_passes=False,
                                             use_tc_tiling_on_sc=False),
        scratch_types=[
            pltpu.VMEM((4 * _SEG,), jnp.float32),
            pltpu.VMEM((_LW,), jnp.int32),
            pltpu.VMEM((_K,), jnp.int32),
            pltpu.VMEM((_K,), jnp.int32),
            pltpu.VMEM((_K,), jnp.int32),
            pltpu.VMEM((_K,), jnp.int32),
            pltpu.VMEM((_K, 16), jnp.float32),
            pltpu.VMEM((_K, 16), jnp.float32),
            pltpu.VMEM((_K, 16), jnp.float32),
            pltpu.VMEM((_K, 16), jnp.float32),
            pltpu.VMEM((96,), jnp.int32),
            pltpu.VMEM((_K,), jnp.int32),
            pltpu.VMEM((_K, _C), jnp.float32),
            pltpu.VMEM((_SEG,), jnp.float32),
            pltpu.SemaphoreType.DMA,
            pltpu.SemaphoreType.DMA,
        ],
    )
    def sck(npc_hbm, lists_hbm, p4v_hbm, ptab_hbm, g_hbm, valid_hbm,
            npcs, listbuf, cidx0, cidx1, cidx2, cidx3,
            candx, candy, candz, candsp, idxbuf, gidx, rows, flags,
            sem, sem2):
        wid = lax.axis_index("s") * _NC + lax.axis_index("c")
        b = wid // (_NW // _B)
        seg = wid % (_NW // _B)
        m0 = seg * _SEG
        bn = b * _N
        bt = b * 4 * _TCH
        for r in range(4):
            pltpu.sync_copy(npc_hbm.at[b * 4 + r, pl.ds(m0, _SEG)],
                            npcs.at[pl.ds(r * _SEG, _SEG)])
        iota = lax.iota(jnp.int32, 16)
        lane0 = iota == 0
        zeros16 = jnp.zeros((16,), jnp.int32)

        def per_centroid(mi, _):
            mi16 = jnp.full((16,), mi, jnp.int32)
            qx = plsc.load_gather(npcs, [mi16])
            qy = plsc.load_gather(npcs, [mi16 + _SEG])
            qz = plsc.load_gather(npcs, [mi16 + 2 * _SEG])
            sq = plsc.load_gather(npcs, [mi16 + 3 * _SEG])
            pltpu.sync_copy(lists_hbm.at[b * _M + m0 + mi], listbuf)
            nproc = listbuf[pl.ds(_K, 16)][0]
            for j in range(_K // 16):
                cid = listbuf[pl.ds(j * 16, 16)]
                cidx0[pl.ds(j * 16, 16)] = cid + bt
                cidx1[pl.ds(j * 16, 16)] = cid + (bt + _TCH)
                cidx2[pl.ds(j * 16, 16)] = cid + (bt + 2 * _TCH)
                cidx3[pl.ds(j * 16, 16)] = cid + (bt + 3 * _TCH)
            cps = [pltpu.async_copy(p4v_hbm.at[ci], cb, sem2)
                   for ci, cb in ((cidx0, candx), (cidx1, candy),
                                  (cidx2, candz), (cidx3, candsp))]
            for cp in cps:
                cp.wait()
            for j in range(6):
                idxbuf[pl.ds(j * 16, 16)] = zeros16

            def chunk(i, tot):
                px = candx[i]
                py = candy[i]
                pz = candz[i]
                sp = candsp[i]
                cid16 = plsc.load_gather(listbuf, [jnp.full((16,), i, jnp.int32)])
                nid = cid16 * 16 + iota
                dot = px * qx + py * qy + pz * qz
                d2 = sq + sp - 2.0 * dot
                msk = d2 < _DIST2
                m32 = msk.astype(jnp.int32)
                r = plsc.cumsum(m32)
                off = jnp.minimum(tot, _K)
                smask = jnp.logical_and(msk, (r + off) <= _K)
                plsc.store_compressed(idxbuf.at[pl.ds(off, 16)],
                                      nid, mask=smask)
                return tot + jnp.sum(m32)

            total = lax.fori_loop(0, nproc, chunk, jnp.int32(0))

            for j in range(_K // 16):
                gidx[pl.ds(j * 16, 16)] = idxbuf[pl.ds(j * 16, 16)] + bn
            pltpu.async_copy(ptab_hbm.at[gidx], rows, sem).wait()
            rowbase = (b * _M + m0 + mi) * _K
            pltpu.sync_copy(rows, g_hbm.at[pl.ds(rowbase, _K)])
            flagv = jnp.where(jnp.full((16,), total) > 0, 1.0, 0.0)
            plsc.store_scatter(flags, [jnp.full((16,), mi, jnp.int32)],
                               flagv, mask=lane0)
            return 0

        lax.fori_loop(0, _SEG, per_centroid, 0)
        pltpu.sync_copy(flags, valid_hbm.at[pl.ds(b * _M + m0, _SEG)])

    return sck(npc4.reshape(_B * 4, _M), lists, p4v, ptab)


def kernel(pc, feat, new_pc, W1, b1, g1, beta1, W2, b2, g2, beta2,
           W3, b3, g3, beta3):
    ptab = _build_point_table(pc, feat)
    G, validf = _sc_query_gather(pc, new_pc, ptab)
    valid = validf.reshape(_B, _M)

    npc32 = jnp.zeros((_B, _M, _C), jnp.float32).at[:, :, :3].set(
        jnp.moveaxis(new_pc, 1, 2))

    return _mlp_passes(G, npc32, valid, W1, b1, g1, beta1,
                       W2, b2, g2, beta2, W3, b3, g3, beta3)


# speedup vs baseline: 2.8910x; 1.1771x over previous
import functools
import math

import jax
import jax.numpy as jnp
from jax import lax
from jax.experimental import pallas as pl
from jax.experimental.pallas import tpu as pltpu
from jax.experimental.pallas import tpu_sc as plsc

_B, _N, _M, _K = 4, 8192, 2048, 64
_INFEA = 16
_DIST2 = 0.4 * 0.4
_EPS = 1e-5
_C = 32
_MB = 128
_PB = _MB * _K
_P_TOT = _B * _M * _K


def _x_tile(g, qpad):
    qb = jnp.broadcast_to(qpad[:, None, :], (_MB, _K, _C)).reshape(_PB, _C)
    return g - qb


def _stats1_kernel(g_ref, npc_ref, acc_ref):
    b = pl.program_id(0)
    mi = pl.program_id(1)

    @pl.when(jnp.logical_and(b == 0, mi == 0))
    def _():
        acc_ref[...] = jnp.zeros_like(acc_ref)

    x = _x_tile(g_ref[...], npc_ref[0])
    gram = jax.lax.dot_general(x, x, (((0,), (0,)), ((), ())),
                               preferred_element_type=jnp.float32)
    s1 = jnp.sum(x, axis=0)
    acc_ref[0:_C, :] += gram
    acc_ref[_C:_C + 1, :] += s1[None, :]


def _stats2_kernel(g_ref, npc_ref, a1_ref, c1_ref, acc_ref):
    b = pl.program_id(0)
    mi = pl.program_id(1)

    @pl.when(jnp.logical_and(b == 0, mi == 0))
    def _():
        acc_ref[...] = jnp.zeros_like(acc_ref)

    x = _x_tile(g_ref[...], npc_ref[0])
    h1 = jnp.maximum(
        jax.lax.dot_general(x, a1_ref[...], (((1,), (1,)), ((), ())),
                            preferred_element_type=jnp.float32)
        + c1_ref[...], 0.0)
    gram = jax.lax.dot_general(h1, h1, (((0,), (0,)), ((), ())),
                               preferred_element_type=jnp.float32)
    acc_ref[0:_C, :] += gram
    acc_ref[_C:_C + 1, :] += jnp.sum(h1, axis=0)[None, :]


def _stats3_kernel(g_ref, npc_ref, a1_ref, c1_ref, a2_ref, c2_ref, acc_ref):
    b = pl.program_id(0)
    mi = pl.program_id(1)

    @pl.when(jnp.logical_and(b == 0, mi == 0))
    def _():
        acc_ref[...] = jnp.zeros_like(acc_ref)

    x = _x_tile(g_ref[...], npc_ref[0])
    h1 = jnp.maximum(
        jax.lax.dot_general(x, a1_ref[...], (((1,), (1,)), ((), ())),
                            preferred_element_type=jnp.float32)
        + c1_ref[...], 0.0)
    h2 = jnp.maximum(
        jax.lax.dot_general(h1, a2_ref[...], (((1,), (1,)), ((), ())),
                            preferred_element_type=jnp.float32)
        + c2_ref[...], 0.0)
    gram = jax.lax.dot_general(h2, h2, (((0,), (0,)), ((), ())),
                               preferred_element_type=jnp.float32)
    acc_ref[0:_C, :] += gram
    acc_ref[_C:_C + 1, :] += jnp.sum(h2, axis=0)[None, :]


def _final_kernel(g_ref, npc_ref, a1_ref, c1_ref, a2_ref, c2_ref,
                  a3_ref, c3_ref, valid_ref, out_ref):
    b = pl.program_id(0)
    x = _x_tile(g_ref[...], npc_ref[0])
    h1 = jnp.maximum(
        jax.lax.dot_general(x, a1_ref[...], (((1,), (1,)), ((), ())),
                            preferred_element_type=jnp.float32)
        + c1_ref[...], 0.0)
    h2 = jnp.maximum(
        jax.lax.dot_general(h1, a2_ref[...], (((1,), (1,)), ((), ())),
                            preferred_element_type=jnp.float32)
        + c2_ref[...], 0.0)
    y = jnp.maximum(
        jax.lax.dot_general(h2, a3_ref[...], (((1,), (1,)), ((), ())),
                            preferred_element_type=jnp.float32)
        + c3_ref[...], 0.0)
    vrow = valid_ref[0, 0]
    vmask = jnp.broadcast_to(vrow.reshape(_MB, 1, 1), (_MB, _K, 1))
    y = y * vmask.reshape(_PB, 1)
    out_ref[0] = y.T.reshape(64, _MB, _K)


def _fold(acc, W, bvec, gvec, beta, cin):
    n = float(_P_TOT)
    gram = acc[0:_C, 0:_C] / n
    mu = acc[_C, 0:_C] / n
    Wp = jnp.zeros((W.shape[0], _C), jnp.float32).at[:, :cin].set(W)
    wmu = Wp @ mu
    mean_y = wmu + bvec
    e_yy = jnp.einsum('oc,cd,od->o', Wp, gram, Wp) + 2.0 * bvec * wmu + bvec * bvec
    var_y = e_yy - mean_y * mean_y
    a = gvec * jax.lax.rsqrt(var_y + _EPS)
    A = a[:, None] * Wp
    c = a * bvec + beta - a * mean_y
    return A, c[None, :]


def _mlp_passes(G, npc32, valid, W1, b1, g1, beta1, W2, b2, g2, beta2,
                W3, b3, g3, beta3):
    valid = valid.reshape(_B, _M // _MB, 1, _MB)
    grid = (_B, _M // _MB)
    g_spec = pl.BlockSpec((_PB, _C), lambda b, mi: (b * (_M // _MB) + mi, 0))
    npc_spec = pl.BlockSpec((1, _MB, _C), lambda b, mi: (b, mi, 0))
    acc_shape = jax.ShapeDtypeStruct((_C + 8, _C), jnp.float32)
    acc_spec = pl.BlockSpec((_C + 8, _C), lambda b, mi: (0, 0))
    mat_spec = pl.BlockSpec((_C, _C), lambda b, mi: (0, 0))
    c_spec = pl.BlockSpec((1, _C), lambda b, mi: (0, 0))

    acc1 = pl.pallas_call(
        _stats1_kernel, grid=grid,
        in_specs=[g_spec, npc_spec],
        out_specs=acc_spec, out_shape=acc_shape,
    )(G, npc32)
    A1, c1 = _fold(acc1, W1, b1, g1, beta1, 3 + _INFEA)

    acc2 = pl.pallas_call(
        _stats2_kernel, grid=grid,
        in_specs=[g_spec, npc_spec, mat_spec, c_spec],
        out_specs=acc_spec, out_shape=acc_shape,
    )(G, npc32, A1, c1)
    A2, c2 = _fold(acc2, W2, b2, g2, beta2, 32)

    acc3 = pl.pallas_call(
        _stats3_kernel, grid=grid,
        in_specs=[g_spec, npc_spec, mat_spec, c_spec, mat_spec, c_spec],
        out_specs=acc_shape and acc_spec, out_shape=acc_shape,
    )(G, npc32, A1, c1, A2, c2)
    A3, c3 = _fold(acc3, W3, b3, g3, beta3, 32)
    A3p = jnp.zeros((64, _C), jnp.float32).at[:, :].set(A3)

    out = pl.pallas_call(
        _final_kernel, grid=grid,
        in_specs=[g_spec, npc_spec, mat_spec, c_spec, mat_spec, c_spec,
                  pl.BlockSpec((64, _C), lambda b, mi: (0, 0)),
                  pl.BlockSpec((1, 64), lambda b, mi: (0, 0)),
                  pl.BlockSpec((1, 1, 1, _MB), lambda b, mi: (b, mi, 0, 0))],
        out_specs=pl.BlockSpec((1, 64, _MB, _K), lambda b, mi: (b, 0, mi, 0)),
        out_shape=jax.ShapeDtypeStruct((_B, 64, _M, _K), jnp.float32),
    )(G, npc32, A1, c1, A2, c2, A3p, c3, valid)
    return out


_NB = 2048


def _prep_kernel(pc_ref, feat_ref, p_ref):
    cat = jnp.concatenate(
        [pc_ref[0], feat_ref[0],
         jnp.zeros((_C - 3 - _INFEA, _NB), jnp.float32)], axis=0)
    p_ref[...] = cat.T


def _build_point_table(pc, feat):
    grid = (_B, _N // _NB)
    return pl.pallas_call(
        _prep_kernel, grid=grid,
        in_specs=[pl.BlockSpec((1, 3, _NB), lambda b, ni: (b, 0, ni)),
                  pl.BlockSpec((1, _INFEA, _NB), lambda b, ni: (b, 0, ni))],
        out_specs=pl.BlockSpec((_NB, _C), lambda b, ni: (b * (_N // _NB) + ni, 0)),
        out_shape=jax.ShapeDtypeStruct((_B * _N, _C), jnp.float32),
    )(pc, feat)


def _pack_kernel(c_ref, out_ref):
    x, y, z = c_ref[0, 0], c_ref[0, 1], c_ref[0, 2]
    r = c_ref[0].astype(jnp.bfloat16).astype(jnp.float32)
    s = (x * x + y * y) + z * z
    out_ref[0] = jnp.concatenate([r, s[None, :]], axis=0)


def _pack4(arr, L):
    nb = min(L, 2048)
    grid = (_B, L // nb)
    return pl.pallas_call(
        _pack_kernel, grid=grid,
        in_specs=[pl.BlockSpec((1, 3, nb), lambda b, ni: (b, 0, ni))],
        out_specs=pl.BlockSpec((1, 4, nb), lambda b, ni: (b, 0, ni)),
        out_shape=jax.ShapeDtypeStruct((_B, 4, L), jnp.float32),
    )(arr)


_TCH = _N // 16
_LW = 80


_LMB = 128


def _list_kernel(npc4_ref, pc4_ref, e2_ref, out_ref):
    q = npc4_ref[0]
    p = pc4_ref[0]
    dot = jax.lax.dot_general(
        q[:3].T.astype(jnp.bfloat16), p[:3].astype(jnp.bfloat16),
        (((1,), (0,)), ((), ())), preferred_element_type=jnp.float32)
    d2 = q[3][:, None] + p[3][None, :] - 2.0 * dot
    mask01 = (d2 < _DIST2).astype(jnp.bfloat16)
    cnts = jax.lax.dot_general(mask01, e2_ref[...],
                               (((1,), (0,)), ((), ())),
                               preferred_element_type=jnp.float32)
    ti = jax.lax.broadcasted_iota(jnp.int32, (_TCH, _TCH), 0)
    tj = jax.lax.broadcasted_iota(jnp.int32, (_TCH, _TCH), 1)
    tri_excl = (ti < tj).astype(jnp.float32)
    tri_incl = (ti <= tj).astype(jnp.float32)
    cum_excl = jax.lax.dot_general(cnts, tri_excl, (((1,), (0,)), ((), ())),
                                   preferred_element_type=jnp.float32)
    nz = (cnts > 0.0).astype(jnp.float32)
    proc = nz * (cum_excl < float(_K)).astype(jnp.float32)
    cum_proc = jax.lax.dot_general(proc, tri_incl, (((1,), (0,)), ((), ())),
                                   preferred_element_type=jnp.float32)
    jslab = 8
    jj0 = jax.lax.broadcasted_iota(jnp.int32, (_LMB, jslab, _TCH), 1).astype(jnp.float32)
    pieces = []
    for jc in range(_K // jslab):
        jj = jj0 + float(jc * jslab)
        pieces.append(jnp.sum((cum_proc[:, None, :] <= jj).astype(jnp.float32),
                              axis=2))
    ids = jnp.concatenate(pieces, axis=1)
    ids = jnp.minimum(ids, float(_TCH - 1)).astype(jnp.int32)
    nproc = jnp.sum(proc, axis=1).astype(jnp.int32)
    pad = jnp.zeros((_LMB, _LW - _K - 1), jnp.int32)
    out_ref[...] = jnp.concatenate([ids, nproc[:, None], pad], axis=1)


def _build_lists(npc4, pc4):
    e2 = (jnp.arange(_N, dtype=jnp.int32)[:, None] // 16
          == jnp.arange(_TCH, dtype=jnp.int32)[None, :]).astype(jnp.bfloat16)
    grid = (_B, _M // _LMB)
    return pl.pallas_call(
        _list_kernel, grid=grid,
        in_specs=[pl.BlockSpec((1, 4, _LMB), lambda b, mi: (b, 0, mi)),
                  pl.BlockSpec((1, 4, _N), lambda b, mi: (b, 0, 0)),
                  pl.BlockSpec((_N, _TCH), lambda b, mi: (0, 0))],
        out_specs=pl.BlockSpec((_LMB, _LW),
                               lambda b, mi: (b * (_M // _LMB) + mi, 0)),
        out_shape=jax.ShapeDtypeStruct((_B * _M, _LW), jnp.int32),
    )(npc4, pc4, e2)


_NC, _NS = 2, 16
_NW = _NC * _NS
_CPW = (_B * _M) // _NW
_SEG = _M // (_NW // _B)
_NCHUNK = _N // 16


def _sc_query_gather(pc, new_pc, ptab):
    npc4 = _pack4(new_pc, _M)
    pc4 = _pack4(pc, _N)
    p4v = pc4.reshape(_B * 4 * _TCH, 16)
    lists = _build_lists(npc4, pc4)
    mesh = plsc.VectorSubcoreMesh(core_axis_name="c", subcore_axis_name="s")

    @functools.partial(
        pl.kernel,
        out_type=(jax.ShapeDtypeStruct((_P_TOT, _C), jnp.float32),
                  jax.ShapeDtypeStruct((_B * _M,), jnp.float32)),
        mesh=mesh,
        compiler_params=pltpu.CompilerParams(needs_layout_passes=False,
                                             use_tc_tiling_on_sc=False),
        scratch_types=[
            pltpu.VMEM((4 * _SEG,), jnp.float32),
            pltpu.VMEM((_LW,), jnp.int32),
            pltpu.VMEM((_K,), jnp.int32),
            pltpu.VMEM((_K,), jnp.int32),
            pltpu.VMEM((_K,), jnp.int32),
            pltpu.VMEM((_K,), jnp.int32),
            pltpu.VMEM((_K, 16), jnp.float32),
            pltpu.VMEM((_K, 16), jnp.float32),
            pltpu.VMEM((_K, 16), jnp.float32),
            pltpu.VMEM((_K, 16), jnp.float32),
            pltpu.VMEM((96,), jnp.int32),
            pltpu.VMEM((_K,), jnp.int32),
            pltpu.VMEM((_K, _C), jnp.float32),
            pltpu.VMEM((_SEG,), jnp.float32),
            pltpu.SemaphoreType.DMA,
            pltpu.SemaphoreType.DMA,
        ],
    )
    def sck(npc_hbm, lists_hbm, p4v_hbm, ptab_hbm, g_hbm, valid_hbm,
            npcs, listbuf, cidx0, cidx1, cidx2, cidx3,
            candx, candy, candz, candsp, idxbuf, gidx, rows, flags,
            sem, sem2):
        wid = lax.axis_index("s") * _NC + lax.axis_index("c")
        b = wid // (_NW // _B)
        seg = wid % (_NW // _B)
        m0 = seg * _SEG
        bn = b * _N
        bt = b * 4 * _TCH
        for r in range(4):
            pltpu.sync_copy(npc_hbm.at[b * 4 + r, pl.ds(m0, _SEG)],
                            npcs.at[pl.ds(r * _SEG, _SEG)])
        iota = lax.iota(jnp.int32, 16)
        lane0 = iota == 0
        zeros16 = jnp.zeros((16,), jnp.int32)

        def per_centroid(mi, _):
            mi16 = jnp.full((16,), mi, jnp.int32)
            qx = plsc.load_gather(npcs, [mi16])
            qy = plsc.load_gather(npcs, [mi16 + _SEG])
            qz = plsc.load_gather(npcs, [mi16 + 2 * _SEG])
            sq = plsc.load_gather(npcs, [mi16 + 3 * _SEG])
            pltpu.sync_copy(lists_hbm.at[b * _M + m0 + mi], listbuf)
            nproc = listbuf[pl.ds(_K, 16)][0]
            for j in range(_K // 16):
                cid = listbuf[pl.ds(j * 16, 16)]
                cidx0[pl.ds(j * 16, 16)] = cid + bt
                cidx1[pl.ds(j * 16, 16)] = cid + (bt + _TCH)
                cidx2[pl.ds(j * 16, 16)] = cid + (bt + 2 * _TCH)
                cidx3[pl.ds(j * 16, 16)] = cid + (bt + 3 * _TCH)
            cps = [pltpu.async_copy(p4v_hbm.at[ci], cb, sem2)
                   for ci, cb in ((cidx0, candx), (cidx1, candy),
                                  (cidx2, candz), (cidx3, candsp))]
            for cp in cps:
                cp.wait()
            for j in range(6):
                idxbuf[pl.ds(j * 16, 16)] = zeros16

            def chunk(i, tot):
                px = candx[i]
                py = candy[i]
                pz = candz[i]
                sp = candsp[i]
                cid16 = plsc.load_gather(listbuf, [jnp.full((16,), i, jnp.int32)])
                nid = cid16 * 16 + iota
                dot = px * qx + py * qy + pz * qz
                d2 = sq + sp - 2.0 * dot
                msk = d2 < _DIST2
                m32 = msk.astype(jnp.int32)
                r = plsc.cumsum(m32)
                off = jnp.minimum(tot, _K)
                smask = jnp.logical_and(msk, (r + off) <= _K)
                plsc.store_compressed(idxbuf.at[pl.ds(off, 16)],
                                      nid, mask=smask)
                return tot + jnp.sum(m32)

            total = lax.fori_loop(0, nproc, chunk, jnp.int32(0))

            for j in range(_K // 16):
                gidx[pl.ds(j * 16, 16)] = idxbuf[pl.ds(j * 16, 16)] + bn
            pltpu.async_copy(ptab_hbm.at[gidx], rows, sem).wait()
            rowbase = (b * _M + m0 + mi) * _K
            pltpu.sync_copy(rows, g_hbm.at[pl.ds(rowbase, _K)])
            flagv = jnp.where(jnp.full((16,), total) > 0, 1.0, 0.0)
            plsc.store_scatter(flags, [jnp.full((16,), mi, jnp.int32)],
                               flagv, mask=lane0)
            return 0

        lax.fori_loop(0, _SEG, per_centroid, 0)
        pltpu.sync_copy(flags, valid_hbm.at[pl.ds(b * _M + m0, _SEG)])

    return sck(npc4.reshape(_B * 4, _M), lists, p4v, ptab)


def kernel(pc, feat, new_pc, W1, b1, g1, beta1, W2, b2, g2, beta2,
           W3, b3, g3, beta3):
    ptab = _build_point_table(pc, feat)
    G, validf = _sc_query_gather(pc, new_pc, ptab)
    valid = validf.reshape(_B, _M)

    npc32 = jnp.zeros((_B, _M, _C), jnp.float32).at[:, :, :3].set(
        jnp.moveaxis(new_pc, 1, 2))

    return _mlp_passes(G, npc32, valid, W1, b1, g1, beta1,
                       W2, b2, g2, beta2, W3, b3, g3, beta3)


# speedup vs baseline: 2.9547x; 1.0221x over previous
import functools
import math

import jax
import jax.numpy as jnp
from jax import lax
from jax.experimental import pallas as pl
from jax.experimental.pallas import tpu as pltpu
from jax.experimental.pallas import tpu_sc as plsc

_B, _N, _M, _K = 4, 8192, 2048, 64
_INFEA = 16
_DIST2 = 0.4 * 0.4
_EPS = 1e-5
_C = 32
_MB = 128
_PB = _MB * _K
_P_TOT = _B * _M * _K


def _x_tile(g, qpad):
    qb = jnp.broadcast_to(qpad[:, None, :], (_MB, _K, _C)).reshape(_PB, _C)
    return g - qb


def _stats1_kernel(g_ref, npc_ref, acc_ref):
    b = pl.program_id(0)
    mi = pl.program_id(1)

    @pl.when(jnp.logical_and(b == 0, mi == 0))
    def _():
        acc_ref[...] = jnp.zeros_like(acc_ref)

    x = _x_tile(g_ref[...], npc_ref[0])
    gram = jax.lax.dot_general(x, x, (((0,), (0,)), ((), ())),
                               preferred_element_type=jnp.float32)
    s1 = jnp.sum(x, axis=0)
    acc_ref[0:_C, :] += gram
    acc_ref[_C:_C + 1, :] += s1[None, :]


def _stats2_kernel(g_ref, npc_ref, a1_ref, c1_ref, acc_ref):
    b = pl.program_id(0)
    mi = pl.program_id(1)

    @pl.when(jnp.logical_and(b == 0, mi == 0))
    def _():
        acc_ref[...] = jnp.zeros_like(acc_ref)

    x = _x_tile(g_ref[...], npc_ref[0])
    h1 = jnp.maximum(
        jax.lax.dot_general(x, a1_ref[...], (((1,), (1,)), ((), ())),
                            preferred_element_type=jnp.float32)
        + c1_ref[...], 0.0)
    gram = jax.lax.dot_general(h1, h1, (((0,), (0,)), ((), ())),
                               preferred_element_type=jnp.float32)
    acc_ref[0:_C, :] += gram
    acc_ref[_C:_C + 1, :] += jnp.sum(h1, axis=0)[None, :]


def _stats3_kernel(g_ref, npc_ref, a1_ref, c1_ref, a2_ref, c2_ref, acc_ref):
    b = pl.program_id(0)
    mi = pl.program_id(1)

    @pl.when(jnp.logical_and(b == 0, mi == 0))
    def _():
        acc_ref[...] = jnp.zeros_like(acc_ref)

    x = _x_tile(g_ref[...], npc_ref[0])
    h1 = jnp.maximum(
        jax.lax.dot_general(x, a1_ref[...], (((1,), (1,)), ((), ())),
                            preferred_element_type=jnp.float32)
        + c1_ref[...], 0.0)
    h2 = jnp.maximum(
        jax.lax.dot_general(h1, a2_ref[...], (((1,), (1,)), ((), ())),
                            preferred_element_type=jnp.float32)
        + c2_ref[...], 0.0)
    gram = jax.lax.dot_general(h2, h2, (((0,), (0,)), ((), ())),
                               preferred_element_type=jnp.float32)
    acc_ref[0:_C, :] += gram
    acc_ref[_C:_C + 1, :] += jnp.sum(h2, axis=0)[None, :]


def _final_kernel(g_ref, npc_ref, a1_ref, c1_ref, a2_ref, c2_ref,
                  a3_ref, c3_ref, valid_ref, out_ref):
    b = pl.program_id(0)
    x = _x_tile(g_ref[...], npc_ref[0])
    h1 = jnp.maximum(
        jax.lax.dot_general(x, a1_ref[...], (((1,), (1,)), ((), ())),
                            preferred_element_type=jnp.float32)
        + c1_ref[...], 0.0)
    h2 = jnp.maximum(
        jax.lax.dot_general(h1, a2_ref[...], (((1,), (1,)), ((), ())),
                            preferred_element_type=jnp.float32)
        + c2_ref[...], 0.0)
    y = jnp.maximum(
        jax.lax.dot_general(h2, a3_ref[...], (((1,), (1,)), ((), ())),
                            preferred_element_type=jnp.float32)
        + c3_ref[...], 0.0)
    vrow = valid_ref[0, 0]
    vmask = jnp.broadcast_to(vrow.reshape(_MB, 1, 1), (_MB, _K, 1))
    y = y * vmask.reshape(_PB, 1)
    out_ref[0] = y.T.reshape(64, _MB, _K)


def _fold(acc, W, bvec, gvec, beta, cin):
    n = float(_P_TOT)
    gram = acc[0:_C, 0:_C] / n
    mu = acc[_C, 0:_C] / n
    Wp = jnp.zeros((W.shape[0], _C), jnp.float32).at[:, :cin].set(W)
    wmu = Wp @ mu
    mean_y = wmu + bvec
    e_yy = jnp.einsum('oc,cd,od->o', Wp, gram, Wp) + 2.0 * bvec * wmu + bvec * bvec
    var_y = e_yy - mean_y * mean_y
    a = gvec * jax.lax.rsqrt(var_y + _EPS)
    A = a[:, None] * Wp
    c = a * bvec + beta - a * mean_y
    return A, c[None, :]


def _mlp_passes(G, npc32, valid, W1, b1, g1, beta1, W2, b2, g2, beta2,
                W3, b3, g3, beta3):
    valid = valid.reshape(_B, _M // _MB, 1, _MB)
    grid = (_B, _M // _MB)
    g_spec = pl.BlockSpec((_PB, _C), lambda b, mi: (b * (_M // _MB) + mi, 0))
    npc_spec = pl.BlockSpec((1, _MB, _C), lambda b, mi: (b, mi, 0))
    acc_shape = jax.ShapeDtypeStruct((_C + 8, _C), jnp.float32)
    acc_spec = pl.BlockSpec((_C + 8, _C), lambda b, mi: (0, 0))
    mat_spec = pl.BlockSpec((_C, _C), lambda b, mi: (0, 0))
    c_spec = pl.BlockSpec((1, _C), lambda b, mi: (0, 0))

    acc1 = pl.pallas_call(
        _stats1_kernel, grid=grid,
        in_specs=[g_spec, npc_spec],
        out_specs=acc_spec, out_shape=acc_shape,
    )(G, npc32)
    A1, c1 = _fold(acc1, W1, b1, g1, beta1, 3 + _INFEA)

    acc2 = pl.pallas_call(
        _stats2_kernel, grid=grid,
        in_specs=[g_spec, npc_spec, mat_spec, c_spec],
        out_specs=acc_spec, out_shape=acc_shape,
    )(G, npc32, A1, c1)
    A2, c2 = _fold(acc2, W2, b2, g2, beta2, 32)

    acc3 = pl.pallas_call(
        _stats3_kernel, grid=grid,
        in_specs=[g_spec, npc_spec, mat_spec, c_spec, mat_spec, c_spec],
        out_specs=acc_shape and acc_spec, out_shape=acc_shape,
    )(G, npc32, A1, c1, A2, c2)
    A3, c3 = _fold(acc3, W3, b3, g3, beta3, 32)
    A3p = jnp.zeros((64, _C), jnp.float32).at[:, :].set(A3)

    out = pl.pallas_call(
        _final_kernel, grid=grid,
        in_specs=[g_spec, npc_spec, mat_spec, c_spec, mat_spec, c_spec,
                  pl.BlockSpec((64, _C), lambda b, mi: (0, 0)),
                  pl.BlockSpec((1, 64), lambda b, mi: (0, 0)),
                  pl.BlockSpec((1, 1, 1, _MB), lambda b, mi: (b, mi, 0, 0))],
        out_specs=pl.BlockSpec((1, 64, _MB, _K), lambda b, mi: (b, 0, mi, 0)),
        out_shape=jax.ShapeDtypeStruct((_B, 64, _M, _K), jnp.float32),
    )(G, npc32, A1, c1, A2, c2, A3p, c3, valid)
    return out


_NB = 2048


def _prep_kernel(pc_ref, feat_ref, p_ref):
    cat = jnp.concatenate(
        [pc_ref[0], feat_ref[0],
         jnp.zeros((_C - 3 - _INFEA, _NB), jnp.float32)], axis=0)
    p_ref[...] = cat.T


def _build_point_table(pc, feat):
    grid = (_B, _N // _NB)
    return pl.pallas_call(
        _prep_kernel, grid=grid,
        in_specs=[pl.BlockSpec((1, 3, _NB), lambda b, ni: (b, 0, ni)),
                  pl.BlockSpec((1, _INFEA, _NB), lambda b, ni: (b, 0, ni))],
        out_specs=pl.BlockSpec((_NB, _C), lambda b, ni: (b * (_N // _NB) + ni, 0)),
        out_shape=jax.ShapeDtypeStruct((_B * _N, _C), jnp.float32),
    )(pc, feat)


def _pack_kernel(c_ref, out_ref):
    x, y, z = c_ref[0, 0], c_ref[0, 1], c_ref[0, 2]
    r = c_ref[0].astype(jnp.bfloat16).astype(jnp.float32)
    s = (x * x + y * y) + z * z
    out_ref[0] = jnp.concatenate([r, s[None, :]], axis=0)


def _pack4(arr, L):
    nb = min(L, 2048)
    grid = (_B, L // nb)
    return pl.pallas_call(
        _pack_kernel, grid=grid,
        in_specs=[pl.BlockSpec((1, 3, nb), lambda b, ni: (b, 0, ni))],
        out_specs=pl.BlockSpec((1, 4, nb), lambda b, ni: (b, 0, ni)),
        out_shape=jax.ShapeDtypeStruct((_B, 4, L), jnp.float32),
    )(arr)


_TCH = _N // 16
_LW = 80


_LMB = 128


def _list_kernel(npc4_ref, pc4_ref, e2_ref, out_ref):
    q = npc4_ref[0]
    p = pc4_ref[0]
    dot = jax.lax.dot_general(
        q[:3].T.astype(jnp.bfloat16), p[:3].astype(jnp.bfloat16),
        (((1,), (0,)), ((), ())), preferred_element_type=jnp.float32)
    d2 = q[3][:, None] + p[3][None, :] - 2.0 * dot
    mask01 = (d2 < _DIST2).astype(jnp.bfloat16)
    cnts = jax.lax.dot_general(mask01, e2_ref[...],
                               (((1,), (0,)), ((), ())),
                               preferred_element_type=jnp.float32)
    ti = jax.lax.broadcasted_iota(jnp.int32, (_TCH, _TCH), 0)
    tj = jax.lax.broadcasted_iota(jnp.int32, (_TCH, _TCH), 1)
    tri_excl = (ti < tj).astype(jnp.float32)
    tri_incl = (ti <= tj).astype(jnp.float32)
    cum_excl = jax.lax.dot_general(cnts, tri_excl, (((1,), (0,)), ((), ())),
                                   preferred_element_type=jnp.float32)
    nz = (cnts > 0.0).astype(jnp.float32)
    proc = nz * (cum_excl < float(_K)).astype(jnp.float32)
    cum_proc = jax.lax.dot_general(proc, tri_incl, (((1,), (0,)), ((), ())),
                                   preferred_element_type=jnp.float32)
    jslab = 8
    jj0 = jax.lax.broadcasted_iota(jnp.int32, (_LMB, jslab, _TCH), 1).astype(jnp.float32)
    pieces = []
    for jc in range(_K // jslab):
        jj = jj0 + float(jc * jslab)
        pieces.append(jnp.sum((cum_proc[:, None, :] <= jj).astype(jnp.float32),
                              axis=2))
    ids = jnp.concatenate(pieces, axis=1)
    ids = jnp.minimum(ids, float(_TCH - 1)).astype(jnp.int32)
    nproc = jnp.sum(proc, axis=1).astype(jnp.int32)
    pad = jnp.zeros((_LMB, _LW - _K - 1), jnp.int32)
    out_ref[...] = jnp.concatenate([ids, nproc[:, None], pad], axis=1)


def _build_lists(npc4, pc4):
    e2 = (jnp.arange(_N, dtype=jnp.int32)[:, None] // 16
          == jnp.arange(_TCH, dtype=jnp.int32)[None, :]).astype(jnp.bfloat16)
    grid = (_B, _M // _LMB)
    return pl.pallas_call(
        _list_kernel, grid=grid,
        in_specs=[pl.BlockSpec((1, 4, _LMB), lambda b, mi: (b, 0, mi)),
                  pl.BlockSpec((1, 4, _N), lambda b, mi: (b, 0, 0)),
                  pl.BlockSpec((_N, _TCH), lambda b, mi: (0, 0))],
        out_specs=pl.BlockSpec((_LMB, _LW),
                               lambda b, mi: (b * (_M // _LMB) + mi, 0)),
        out_shape=jax.ShapeDtypeStruct((_B * _M, _LW), jnp.int32),
    )(npc4, pc4, e2)


_NC, _NS = 2, 16
_NW = _NC * _NS
_CPW = (_B * _M) // _NW
_SEG = _M // (_NW // _B)
_NCHUNK = _N // 16


def _sc_query_gather(pc, new_pc, ptab):
    npc4 = _pack4(new_pc, _M)
    pc4 = _pack4(pc, _N)
    p4v = pc4.reshape(_B * 4 * _TCH, 16)
    lists = _build_lists(npc4, pc4)
    mesh = plsc.VectorSubcoreMesh(core_axis_name="c", subcore_axis_name="s")

    @functools.partial(
        pl.kernel,
        out_type=(jax.ShapeDtypeStruct((_P_TOT, _C), jnp.float32),
                  jax.ShapeDtypeStruct((_B * _M,), jnp.float32)),
        mesh=mesh,
        compiler_params=pltpu.CompilerParams(needs_layout_passes=False,
                                             use_tc_tiling_on_sc=False),
        scratch_types=[
            pltpu.VMEM((4 * _SEG,), jnp.float32),
            pltpu.VMEM((2 * _LW,), jnp.int32),
            pltpu.VMEM((2 * 4 * _K,), jnp.int32),
            pltpu.VMEM((2 * 4 * _K, 16), jnp.float32),
            pltpu.VMEM((96,), jnp.int32),
            pltpu.VMEM((_K,), jnp.int32),
            pltpu.VMEM((2 * _K, _C), jnp.float32),
            pltpu.VMEM((_SEG,), jnp.float32),
            pltpu.SemaphoreType.DMA,
            pltpu.SemaphoreType.DMA,
            pltpu.SemaphoreType.DMA,
            pltpu.SemaphoreType.DMA,
            pltpu.SemaphoreType.DMA,
            pltpu.SemaphoreType.DMA,
            pltpu.SemaphoreType.DMA,
        ],
    )
    def sck(npc_hbm, lists_hbm, p4v_hbm, ptab_hbm, g_hbm, valid_hbm,
            npcs, listbuf, cidx, cand, idxbuf, gidx, rows, flags,
            semr, semc0, semc1, seml0, seml1, sems0, sems1):
        wid = lax.axis_index("s") * _NC + lax.axis_index("c")
        b = wid // (_NW // _B)
        seg = wid % (_NW // _B)
        m0 = seg * _SEG
        bn = b * _N
        bt = b * 4 * _TCH
        for r in range(4):
            pltpu.sync_copy(npc_hbm.at[b * 4 + r, pl.ds(m0, _SEG)],
                            npcs.at[pl.ds(r * _SEG, _SEG)])
        iota = lax.iota(jnp.int32, 16)
        lane0 = iota == 0
        zeros16 = jnp.zeros((16,), jnp.int32)
        semc = (semc0, semc1)
        seml = (seml0, seml1)
        sems = (sems0, sems1)

        def lrow(mi):
            return lists_hbm.at[b * _M + m0 + jnp.minimum(mi, _SEG - 1)]

        def lslice(par):
            return listbuf.at[pl.ds(par * _LW, _LW)]

        def fire_chunks(par):
            for j in range(_K // 16):
                cid = listbuf[pl.ds(par * _LW + j * 16, 16)]
                for comp in range(4):
                    cidx[pl.ds(par * 4 * _K + comp * _K + j * 16, 16)] = (
                        cid + (bt + comp * _TCH))
            for comp in range(4):
                pltpu.async_copy(
                    p4v_hbm.at[cidx.at[pl.ds(par * 4 * _K + comp * _K, _K)]],
                    cand.at[pl.ds(par * 4 * _K + comp * _K, _K)], semc[par])

        def wait_chunks(par):
            for comp in range(4):
                pltpu.make_async_copy(
                    p4v_hbm.at[cidx.at[pl.ds(par * 4 * _K + comp * _K, _K)]],
                    cand.at[pl.ds(par * 4 * _K + comp * _K, _K)],
                    semc[par]).wait()

        pltpu.sync_copy(lrow(jnp.int32(0)), lslice(0))
        fire_chunks(0)
        pltpu.async_copy(lrow(jnp.int32(1)), lslice(1), seml[1])

        def process(g, par):
            mi = 2 * g + par
            npar = 1 - par
            pltpu.make_async_copy(lrow(mi + 1), lslice(npar), seml[npar]).wait()
            fire_chunks(npar)
            wait_chunks(par)
            mi16 = jnp.full((16,), mi, jnp.int32)
            qx = plsc.load_gather(npcs, [mi16])
            qy = plsc.load_gather(npcs, [mi16 + _SEG])
            qz = plsc.load_gather(npcs, [mi16 + 2 * _SEG])
            sq = plsc.load_gather(npcs, [mi16 + 3 * _SEG])
            nproc = listbuf[pl.ds(par * _LW + _K, 16)][0]
            for j in range(6):
                idxbuf[pl.ds(j * 16, 16)] = zeros16

            def chunk(i, tot):
                base = par * 4 * _K
                px = cand[base + i]
                py = cand[base + _K + i]
                pz = cand[base + 2 * _K + i]
                sp = cand[base + 3 * _K + i]
                cid16 = plsc.load_gather(
                    listbuf, [jnp.full((16,), par * _LW + i, jnp.int32)])
                nid = cid16 * 16 + iota
                dot = px * qx + py * qy + pz * qz
                d2 = sq + sp - 2.0 * dot
                msk = d2 < _DIST2
                m32 = msk.astype(jnp.int32)
                r = plsc.cumsum(m32)
                off = jnp.minimum(tot, _K)
                smask = jnp.logical_and(msk, (r + off) <= _K)
                plsc.store_compressed(idxbuf.at[pl.ds(off, 16)],
                                      nid, mask=smask)
                return tot + jnp.sum(m32)

            total = lax.fori_loop(0, nproc, chunk, jnp.int32(0))

            pltpu.async_copy(lrow(mi + 2), lslice(par), seml[par])

            for j in range(_K // 16):
                gidx[pl.ds(j * 16, 16)] = idxbuf[pl.ds(j * 16, 16)] + bn
            rslice = rows.at[pl.ds(par * _K, _K)]
            rowbase = (b * _M + m0 + mi) * _K
            gdst = g_hbm.at[pl.ds(rowbase, _K)]

            @pl.when(g > 0)
            def _():
                pltpu.make_async_copy(
                    rslice, g_hbm.at[pl.ds((b * _M + m0 + mi - 2) * _K, _K)],
                    sems[par]).wait()

            pltpu.async_copy(ptab_hbm.at[gidx], rslice, semr).wait()
            pltpu.async_copy(rslice, gdst, sems[par])
            flagv = jnp.where(jnp.full((16,), total) > 0, 1.0, 0.0)
            plsc.store_scatter(flags, [mi16], flagv, mask=lane0)

        def body(g, _):
            process(g, 0)
            process(g, 1)
            return 0

        lax.fori_loop(0, _SEG // 2, body, 0)
        wait_chunks(0)
        pltpu.make_async_copy(lrow(jnp.int32(_SEG - 1)), lslice(1),
                              seml[1]).wait()
        for par in range(2):
            pltpu.make_async_copy(
                rows.at[pl.ds(par * _K, _K)],
                g_hbm.at[pl.ds((b * _M + m0 + _SEG - 2 + par) * _K, _K)],
                sems[par]).wait()
        pltpu.sync_copy(flags, valid_hbm.at[pl.ds(b * _M + m0, _SEG)])

    return sck(npc4.reshape(_B * 4, _M), lists, p4v, ptab)


def kernel(pc, feat, new_pc, W1, b1, g1, beta1, W2, b2, g2, beta2,
           W3, b3, g3, beta3):
    ptab = _build_point_table(pc, feat)
    G, validf = _sc_query_gather(pc, new_pc, ptab)
    valid = validf.reshape(_B, _M)

    npc32 = jnp.zeros((_B, _M, _C), jnp.float32).at[:, :, :3].set(
        jnp.moveaxis(new_pc, 1, 2))

    return _mlp_passes(G, npc32, valid, W1, b1, g1, beta1,
                       W2, b2, g2, beta2, W3, b3, g3, beta3)


# speedup vs baseline: 3.0271x; 1.0245x over previous
import functools
import math

import jax
import jax.numpy as jnp
from jax import lax
from jax.experimental import pallas as pl
from jax.experimental.pallas import tpu as pltpu
from jax.experimental.pallas import tpu_sc as plsc

_B, _N, _M, _K = 4, 8192, 2048, 64
_INFEA = 16
_DIST2 = 0.4 * 0.4
_EPS = 1e-5
_C = 32
_MB = 256
_PB = _MB * _K
_P_TOT = _B * _M * _K


def _x_tile(g, qpad):
    qb = jnp.broadcast_to(qpad[:, None, :], (_MB, _K, _C)).reshape(_PB, _C)
    return g - qb


def _stats1_kernel(g_ref, npc_ref, acc_ref):
    b = pl.program_id(0)
    mi = pl.program_id(1)

    @pl.when(jnp.logical_and(b == 0, mi == 0))
    def _():
        acc_ref[...] = jnp.zeros_like(acc_ref)

    x = _x_tile(g_ref[...], npc_ref[0])
    gram = jax.lax.dot_general(x, x, (((0,), (0,)), ((), ())),
                               preferred_element_type=jnp.float32)
    s1 = jnp.sum(x, axis=0)
    acc_ref[0:_C, :] += gram
    acc_ref[_C:_C + 1, :] += s1[None, :]


def _stats2_kernel(g_ref, npc_ref, a1_ref, c1_ref, acc_ref):
    b = pl.program_id(0)
    mi = pl.program_id(1)

    @pl.when(jnp.logical_and(b == 0, mi == 0))
    def _():
        acc_ref[...] = jnp.zeros_like(acc_ref)

    x = _x_tile(g_ref[...], npc_ref[0])
    h1 = jnp.maximum(
        jax.lax.dot_general(x, a1_ref[...], (((1,), (1,)), ((), ())),
                            preferred_element_type=jnp.float32)
        + c1_ref[...], 0.0)
    gram = jax.lax.dot_general(h1, h1, (((0,), (0,)), ((), ())),
                               preferred_element_type=jnp.float32)
    acc_ref[0:_C, :] += gram
    acc_ref[_C:_C + 1, :] += jnp.sum(h1, axis=0)[None, :]


def _stats3_kernel(g_ref, npc_ref, a1_ref, c1_ref, a2_ref, c2_ref, acc_ref):
    b = pl.program_id(0)
    mi = pl.program_id(1)

    @pl.when(jnp.logical_and(b == 0, mi == 0))
    def _():
        acc_ref[...] = jnp.zeros_like(acc_ref)

    x = _x_tile(g_ref[...], npc_ref[0])
    h1 = jnp.maximum(
        jax.lax.dot_general(x, a1_ref[...], (((1,), (1,)), ((), ())),
                            preferred_element_type=jnp.float32)
        + c1_ref[...], 0.0)
    h2 = jnp.maximum(
        jax.lax.dot_general(h1, a2_ref[...], (((1,), (1,)), ((), ())),
                            preferred_element_type=jnp.float32)
        + c2_ref[...], 0.0)
    gram = jax.lax.dot_general(h2, h2, (((0,), (0,)), ((), ())),
                               preferred_element_type=jnp.float32)
    acc_ref[0:_C, :] += gram
    acc_ref[_C:_C + 1, :] += jnp.sum(h2, axis=0)[None, :]


def _final_kernel(g_ref, npc_ref, a1_ref, c1_ref, a2_ref, c2_ref,
                  a3_ref, c3_ref, valid_ref, out_ref):
    b = pl.program_id(0)
    x = _x_tile(g_ref[...], npc_ref[0])
    h1 = jnp.maximum(
        jax.lax.dot_general(x, a1_ref[...], (((1,), (1,)), ((), ())),
                            preferred_element_type=jnp.float32)
        + c1_ref[...], 0.0)
    h2 = jnp.maximum(
        jax.lax.dot_general(h1, a2_ref[...], (((1,), (1,)), ((), ())),
                            preferred_element_type=jnp.float32)
        + c2_ref[...], 0.0)
    y = jnp.maximum(
        jax.lax.dot_general(h2, a3_ref[...], (((1,), (1,)), ((), ())),
                            preferred_element_type=jnp.float32)
        + c3_ref[...], 0.0)
    vrow = valid_ref[0, 0]
    vmask = jnp.broadcast_to(vrow.reshape(_MB, 1, 1), (_MB, _K, 1))
    y = y * vmask.reshape(_PB, 1)
    out_ref[0] = y.T.reshape(64, _MB, _K)


def _fold(acc, W, bvec, gvec, beta, cin):
    n = float(_P_TOT)
    gram = acc[0:_C, 0:_C] / n
    mu = acc[_C, 0:_C] / n
    Wp = jnp.zeros((W.shape[0], _C), jnp.float32).at[:, :cin].set(W)
    wmu = Wp @ mu
    mean_y = wmu + bvec
    e_yy = jnp.einsum('oc,cd,od->o', Wp, gram, Wp) + 2.0 * bvec * wmu + bvec * bvec
    var_y = e_yy - mean_y * mean_y
    a = gvec * jax.lax.rsqrt(var_y + _EPS)
    A = a[:, None] * Wp
    c = a * bvec + beta - a * mean_y
    return A, c[None, :]


def _mlp_passes(G, npc32, valid, W1, b1, g1, beta1, W2, b2, g2, beta2,
                W3, b3, g3, beta3):
    valid = valid.reshape(_B, _M // _MB, 1, _MB)
    grid = (_B, _M // _MB)
    g_spec = pl.BlockSpec((_PB, _C), lambda b, mi: (b * (_M // _MB) + mi, 0))
    npc_spec = pl.BlockSpec((1, _MB, _C), lambda b, mi: (b, mi, 0))
    acc_shape = jax.ShapeDtypeStruct((_C + 8, _C), jnp.float32)
    acc_spec = pl.BlockSpec((_C + 8, _C), lambda b, mi: (0, 0))
    mat_spec = pl.BlockSpec((_C, _C), lambda b, mi: (0, 0))
    c_spec = pl.BlockSpec((1, _C), lambda b, mi: (0, 0))

    acc1 = pl.pallas_call(
        _stats1_kernel, grid=grid,
        in_specs=[g_spec, npc_spec],
        out_specs=acc_spec, out_shape=acc_shape,
    )(G, npc32)
    A1, c1 = _fold(acc1, W1, b1, g1, beta1, 3 + _INFEA)

    acc2 = pl.pallas_call(
        _stats2_kernel, grid=grid,
        in_specs=[g_spec, npc_spec, mat_spec, c_spec],
        out_specs=acc_spec, out_shape=acc_shape,
    )(G, npc32, A1, c1)
    A2, c2 = _fold(acc2, W2, b2, g2, beta2, 32)

    acc3 = pl.pallas_call(
        _stats3_kernel, grid=grid,
        in_specs=[g_spec, npc_spec, mat_spec, c_spec, mat_spec, c_spec],
        out_specs=acc_shape and acc_spec, out_shape=acc_shape,
    )(G, npc32, A1, c1, A2, c2)
    A3, c3 = _fold(acc3, W3, b3, g3, beta3, 32)
    A3p = jnp.zeros((64, _C), jnp.float32).at[:, :].set(A3)

    out = pl.pallas_call(
        _final_kernel, grid=grid,
        in_specs=[g_spec, npc_spec, mat_spec, c_spec, mat_spec, c_spec,
                  pl.BlockSpec((64, _C), lambda b, mi: (0, 0)),
                  pl.BlockSpec((1, 64), lambda b, mi: (0, 0)),
                  pl.BlockSpec((1, 1, 1, _MB), lambda b, mi: (b, mi, 0, 0))],
        out_specs=pl.BlockSpec((1, 64, _MB, _K), lambda b, mi: (b, 0, mi, 0)),
        out_shape=jax.ShapeDtypeStruct((_B, 64, _M, _K), jnp.float32),
    )(G, npc32, A1, c1, A2, c2, A3p, c3, valid)
    return out


_NB = 2048


def _prep_kernel(pc_ref, feat_ref, p_ref):
    cat = jnp.concatenate(
        [pc_ref[0], feat_ref[0],
         jnp.zeros((_C - 3 - _INFEA, _NB), jnp.float32)], axis=0)
    p_ref[...] = cat.T


def _build_point_table(pc, feat):
    grid = (_B, _N // _NB)
    return pl.pallas_call(
        _prep_kernel, grid=grid,
        in_specs=[pl.BlockSpec((1, 3, _NB), lambda b, ni: (b, 0, ni)),
                  pl.BlockSpec((1, _INFEA, _NB), lambda b, ni: (b, 0, ni))],
        out_specs=pl.BlockSpec((_NB, _C), lambda b, ni: (b * (_N // _NB) + ni, 0)),
        out_shape=jax.ShapeDtypeStruct((_B * _N, _C), jnp.float32),
    )(pc, feat)


def _pack_kernel(c_ref, out_ref):
    x, y, z = c_ref[0, 0], c_ref[0, 1], c_ref[0, 2]
    r = c_ref[0].astype(jnp.bfloat16).astype(jnp.float32)
    s = (x * x + y * y) + z * z
    out_ref[0] = jnp.concatenate([r, s[None, :]], axis=0)


def _pack4(arr, L):
    nb = min(L, 2048)
    grid = (_B, L // nb)
    return pl.pallas_call(
        _pack_kernel, grid=grid,
        in_specs=[pl.BlockSpec((1, 3, nb), lambda b, ni: (b, 0, ni))],
        out_specs=pl.BlockSpec((1, 4, nb), lambda b, ni: (b, 0, ni)),
        out_shape=jax.ShapeDtypeStruct((_B, 4, L), jnp.float32),
    )(arr)


_TCH = _N // 16
_LW = 80


_LMB = 128


def _list_kernel(npc4_ref, pc4_ref, e2_ref, out_ref):
    q = npc4_ref[0]
    p = pc4_ref[0]
    dot = jax.lax.dot_general(
        q[:3].T.astype(jnp.bfloat16), p[:3].astype(jnp.bfloat16),
        (((1,), (0,)), ((), ())), preferred_element_type=jnp.float32)
    d2 = q[3][:, None] + p[3][None, :] - 2.0 * dot
    mask01 = (d2 < _DIST2).astype(jnp.bfloat16)
    cnts = jax.lax.dot_general(mask01, e2_ref[...],
                               (((1,), (0,)), ((), ())),
                               preferred_element_type=jnp.float32)
    ti = jax.lax.broadcasted_iota(jnp.int32, (_TCH, _TCH), 0)
    tj = jax.lax.broadcasted_iota(jnp.int32, (_TCH, _TCH), 1)
    tri_excl = (ti < tj).astype(jnp.float32)
    tri_incl = (ti <= tj).astype(jnp.float32)
    cum_excl = jax.lax.dot_general(cnts, tri_excl, (((1,), (0,)), ((), ())),
                                   preferred_element_type=jnp.float32)
    nz = (cnts > 0.0).astype(jnp.float32)
    proc = nz * (cum_excl < float(_K)).astype(jnp.float32)
    cum_proc = jax.lax.dot_general(proc, tri_incl, (((1,), (0,)), ((), ())),
                                   preferred_element_type=jnp.float32)
    jslab = 8
    jj0 = jax.lax.broadcasted_iota(jnp.int32, (_LMB, jslab, _TCH), 1).astype(jnp.float32)
    pieces = []
    for jc in range(_K // jslab):
        jj = jj0 + float(jc * jslab)
        pieces.append(jnp.sum((cum_proc[:, None, :] <= jj).astype(jnp.float32),
                              axis=2))
    ids = jnp.concatenate(pieces, axis=1)
    ids = jnp.minimum(ids, float(_TCH - 1)).astype(jnp.int32)
    nproc = jnp.sum(proc, axis=1).astype(jnp.int32)
    pad = jnp.zeros((_LMB, _LW - _K - 1), jnp.int32)
    out_ref[...] = jnp.concatenate([ids, nproc[:, None], pad], axis=1)


def _build_lists(npc4, pc4):
    e2 = (jnp.arange(_N, dtype=jnp.int32)[:, None] // 16
          == jnp.arange(_TCH, dtype=jnp.int32)[None, :]).astype(jnp.bfloat16)
    grid = (_B, _M // _LMB)
    return pl.pallas_call(
        _list_kernel, grid=grid,
        in_specs=[pl.BlockSpec((1, 4, _LMB), lambda b, mi: (b, 0, mi)),
                  pl.BlockSpec((1, 4, _N), lambda b, mi: (b, 0, 0)),
                  pl.BlockSpec((_N, _TCH), lambda b, mi: (0, 0))],
        out_specs=pl.BlockSpec((_LMB, _LW),
                               lambda b, mi: (b * (_M // _LMB) + mi, 0)),
        out_shape=jax.ShapeDtypeStruct((_B * _M, _LW), jnp.int32),
    )(npc4, pc4, e2)


_NC, _NS = 2, 16
_NW = _NC * _NS
_CPW = (_B * _M) // _NW
_SEG = _M // (_NW // _B)
_NCHUNK = _N // 16


def _sc_query_gather(pc, new_pc, ptab):
    npc4 = _pack4(new_pc, _M)
    pc4 = _pack4(pc, _N)
    p4v = pc4.reshape(_B * 4 * _TCH, 16)
    lists = _build_lists(npc4, pc4)
    mesh = plsc.VectorSubcoreMesh(core_axis_name="c", subcore_axis_name="s")

    @functools.partial(
        pl.kernel,
        out_type=(jax.ShapeDtypeStruct((_P_TOT, _C), jnp.float32),
                  jax.ShapeDtypeStruct((_B * _M,), jnp.float32)),
        mesh=mesh,
        compiler_params=pltpu.CompilerParams(needs_layout_passes=False,
                                             use_tc_tiling_on_sc=False),
        scratch_types=[
            pltpu.VMEM((4 * _SEG,), jnp.float32),
            pltpu.VMEM((2 * _LW,), jnp.int32),
            pltpu.VMEM((2 * 4 * _K,), jnp.int32),
            pltpu.VMEM((2 * 4 * _K, 16), jnp.float32),
            pltpu.VMEM((96,), jnp.int32),
            pltpu.VMEM((_K,), jnp.int32),
            pltpu.VMEM((2 * _K, _C), jnp.float32),
            pltpu.VMEM((_SEG,), jnp.float32),
            pltpu.SemaphoreType.DMA,
            pltpu.SemaphoreType.DMA,
            pltpu.SemaphoreType.DMA,
            pltpu.SemaphoreType.DMA,
            pltpu.SemaphoreType.DMA,
            pltpu.SemaphoreType.DMA,
            pltpu.SemaphoreType.DMA,
        ],
    )
    def sck(npc_hbm, lists_hbm, p4v_hbm, ptab_hbm, g_hbm, valid_hbm,
            npcs, listbuf, cidx, cand, idxbuf, gidx, rows, flags,
            semr, semc0, semc1, seml0, seml1, sems0, sems1):
        wid = lax.axis_index("s") * _NC + lax.axis_index("c")
        b = wid // (_NW // _B)
        seg = wid % (_NW // _B)
        m0 = seg * _SEG
        bn = b * _N
        bt = b * 4 * _TCH
        for r in range(4):
            pltpu.sync_copy(npc_hbm.at[b * 4 + r, pl.ds(m0, _SEG)],
                            npcs.at[pl.ds(r * _SEG, _SEG)])
        iota = lax.iota(jnp.int32, 16)
        lane0 = iota == 0
        zeros16 = jnp.zeros((16,), jnp.int32)
        semc = (semc0, semc1)
        seml = (seml0, seml1)
        sems = (sems0, sems1)

        def lrow(mi):
            return lists_hbm.at[b * _M + m0 + jnp.minimum(mi, _SEG - 1)]

        def lslice(par):
            return listbuf.at[pl.ds(par * _LW, _LW)]

        def fire_chunks(par):
            for j in range(_K // 16):
                cid = listbuf[pl.ds(par * _LW + j * 16, 16)]
                for comp in range(4):
                    cidx[pl.ds(par * 4 * _K + comp * _K + j * 16, 16)] = (
                        cid + (bt + comp * _TCH))
            for comp in range(4):
                pltpu.async_copy(
                    p4v_hbm.at[cidx.at[pl.ds(par * 4 * _K + comp * _K, _K)]],
                    cand.at[pl.ds(par * 4 * _K + comp * _K, _K)], semc[par])

        def wait_chunks(par):
            for comp in range(4):
                pltpu.make_async_copy(
                    p4v_hbm.at[cidx.at[pl.ds(par * 4 * _K + comp * _K, _K)]],
                    cand.at[pl.ds(par * 4 * _K + comp * _K, _K)],
                    semc[par]).wait()

        pltpu.sync_copy(lrow(jnp.int32(0)), lslice(0))
        fire_chunks(0)
        pltpu.async_copy(lrow(jnp.int32(1)), lslice(1), seml[1])

        def process(g, par):
            mi = 2 * g + par
            npar = 1 - par
            pltpu.make_async_copy(lrow(mi + 1), lslice(npar), seml[npar]).wait()
            fire_chunks(npar)
            wait_chunks(par)
            mi16 = jnp.full((16,), mi, jnp.int32)
            qx = plsc.load_gather(npcs, [mi16])
            qy = plsc.load_gather(npcs, [mi16 + _SEG])
            qz = plsc.load_gather(npcs, [mi16 + 2 * _SEG])
            sq = plsc.load_gather(npcs, [mi16 + 3 * _SEG])
            nproc = listbuf[pl.ds(par * _LW + _K, 16)][0]
            for j in range(6):
                idxbuf[pl.ds(j * 16, 16)] = zeros16

            def chunk(i, tot):
                base = par * 4 * _K
                px = cand[base + i]
                py = cand[base + _K + i]
                pz = cand[base + 2 * _K + i]
                sp = cand[base + 3 * _K + i]
                cid16 = plsc.load_gather(
                    listbuf, [jnp.full((16,), par * _LW + i, jnp.int32)])
                nid = cid16 * 16 + iota
                dot = px * qx + py * qy + pz * qz
                d2 = sq + sp - 2.0 * dot
                msk = d2 < _DIST2
                m32 = msk.astype(jnp.int32)
                r = plsc.cumsum(m32)
                off = jnp.minimum(tot, _K)
                smask = jnp.logical_and(msk, (r + off) <= _K)
                plsc.store_compressed(idxbuf.at[pl.ds(off, 16)],
                                      nid, mask=smask)
                return tot + jnp.sum(m32)

            total = lax.fori_loop(0, nproc, chunk, jnp.int32(0))

            pltpu.async_copy(lrow(mi + 2), lslice(par), seml[par])

            for j in range(_K // 16):
                gidx[pl.ds(j * 16, 16)] = idxbuf[pl.ds(j * 16, 16)] + bn
            rslice = rows.at[pl.ds(par * _K, _K)]
            rowbase = (b * _M + m0 + mi) * _K
            gdst = g_hbm.at[pl.ds(rowbase, _K)]

            @pl.when(g > 0)
            def _():
                pltpu.make_async_copy(
                    rslice, g_hbm.at[pl.ds((b * _M + m0 + mi - 2) * _K, _K)],
                    sems[par]).wait()

            pltpu.async_copy(ptab_hbm.at[gidx], rslice, semr).wait()
            pltpu.async_copy(rslice, gdst, sems[par])
            flagv = jnp.where(jnp.full((16,), total) > 0, 1.0, 0.0)
            plsc.store_scatter(flags, [mi16], flagv, mask=lane0)

        def body(g, _):
            process(g, 0)
            process(g, 1)
            return 0

        lax.fori_loop(0, _SEG // 2, body, 0)
        wait_chunks(0)
        pltpu.make_async_copy(lrow(jnp.int32(_SEG - 1)), lslice(1),
                              seml[1]).wait()
        for par in range(2):
            pltpu.make_async_copy(
                rows.at[pl.ds(par * _K, _K)],
                g_hbm.at[pl.ds((b * _M + m0 + _SEG - 2 + par) * _K, _K)],
                sems[par]).wait()
        pltpu.sync_copy(flags, valid_hbm.at[pl.ds(b * _M + m0, _SEG)])

    return sck(npc4.reshape(_B * 4, _M), lists, p4v, ptab)


def kernel(pc, feat, new_pc, W1, b1, g1, beta1, W2, b2, g2, beta2,
           W3, b3, g3, beta3):
    ptab = _build_point_table(pc, feat)
    G, validf = _sc_query_gather(pc, new_pc, ptab)
    valid = validf.reshape(_B, _M)

    npc32 = jnp.zeros((_B, _M, _C), jnp.float32).at[:, :, :3].set(
        jnp.moveaxis(new_pc, 1, 2))

    return _mlp_passes(G, npc32, valid, W1, b1, g1, beta1,
                       W2, b2, g2, beta2, W3, b3, g3, beta3)
